# baseline (speedup 1.0000x reference)
import functools

import jax
import jax.numpy as jnp
from jax import lax
from jax.experimental import pallas as pl
from jax.experimental.pallas import tpu as pltpu
from jax.experimental.pallas import tpu_sc as plsc

N = 10000
E = 320000
IN = 128
H = 256
HH = 128
OUT = 40

NC = 2
NS = 16
NPAD = 10240
DUMP = NPAD - 1
CHUNK = 256
G = 64
NSL = CHUNK // G
DG = 64
NGR = CHUNK // DG
E2 = 327680
NCHT = E2 // CHUNK
CPT_COL = NCHT // NS
CPT_HALF = NCHT // (NC * NS)
RT = NPAD // NS

_MESH = dict(core_axis_name="c", subcore_axis_name="s")


def _prop_body(tab, se, de, acc, out_c, sst, dst_, sidx, didx, buf,
               stsem, gsem, ssem, base, cpt, s):
    for b in range(2):
        pltpu.async_copy(se.at[base + b], sst.at[b], stsem.at[b])
        pltpu.async_copy(de.at[base + b], dst_.at[b], stsem.at[b])

    def chunk2(i, carry):
        for b in range(2):
            ch = i * 2 + b
            pltpu.make_async_copy(se.at[base + b], sst.at[b], stsem.at[b]).wait()
            pltpu.make_async_copy(de.at[base + b], dst_.at[b], stsem.at[b]).wait()

            for r in range(NSL):
                ds_ = b * NSL + r
                for j in range(G // 16):
                    sidx[r, pl.ds(j * 16, 16)] = sst[b, 0, pl.ds(r * G + j * 16, 16)]
                    didx[ds_, pl.ds(j * 16, 16)] = dst_[b, 0, pl.ds(r * G + j * 16, 16)]

                @pl.when(ch > 0)
                def _():
                    pltpu.make_async_copy(
                        buf.at[r], acc.at[didx.at[ds_]], ssem.at[r]).wait()

                pltpu.async_copy(tab.at[sidx.at[r]], buf.at[r], gsem.at[r])

            @pl.when(ch + 2 < cpt)
            def _():
                pltpu.async_copy(se.at[base + ch + 2], sst.at[b], stsem.at[b])
                pltpu.async_copy(de.at[base + ch + 2], dst_.at[b], stsem.at[b])

            for r in range(NSL):
                ds_ = b * NSL + r
                pltpu.make_async_copy(tab.at[sidx.at[r]], buf.at[r], gsem.at[r]).wait()
                pltpu.async_copy(buf.at[r], acc.at[didx.at[ds_]], ssem.at[r], add=True)
        return carry

    lax.fori_loop(0, cpt // 2, chunk2, 0)
    for r in range(NSL):
        pltpu.make_async_copy(buf.at[r], acc.at[didx.at[NSL + r]], ssem.at[r]).wait()
    plsc.subcore_barrier()
    pltpu.sync_copy(acc.at[pl.ds(s * RT, RT)], out_c.at[pl.ds(s * RT, RT)])


def _prop_scratch():
    return [
        pltpu.VMEM((2, 1, CHUNK), jnp.int32),
        pltpu.VMEM((2, 1, CHUNK), jnp.int32),
        pltpu.VMEM((NSL, G), jnp.int32),
        pltpu.VMEM((2 * NSL, G), jnp.int32),
        pltpu.VMEM((NSL, G, HH), jnp.float32),
        pltpu.VMEM_SHARED((NPAD, HH), jnp.float32),
        pltpu.SemaphoreType.DMA((2,)),
        pltpu.SemaphoreType.DMA((NSL,)),
        pltpu.SemaphoreType.DMA((NSL,)),
    ]


@functools.partial(
    pl.kernel,
    out_type=jax.ShapeDtypeStruct((NC, NPAD, HH), jnp.float32),
    mesh=plsc.VectorSubcoreMesh(**_MESH),
    scratch_types=_prop_scratch(),
)
def _prop_col(hs2, se, de, z, out, sst, dst_, sidx, didx, buf, acc,
              stsem, gsem, ssem):
    c = lax.axis_index("c")
    s = lax.axis_index("s")
    pltpu.sync_copy(z, acc.at[pl.ds(s * RT, RT)])
    plsc.subcore_barrier()
    _prop_body(hs2.at[c], se, de, acc, out.at[c], sst, dst_, sidx, didx, buf,
               stsem, gsem, ssem, s * CPT_COL, CPT_COL, s)


@functools.partial(
    pl.kernel,
    out_type=jax.ShapeDtypeStruct((NC, NPAD, HH), jnp.float32),
    mesh=plsc.VectorSubcoreMesh(**_MESH),
    scratch_types=_prop_scratch(),
)
def _prop_half(hs3, se, de, z, out, sst, dst_, sidx, didx, buf, acc,
               stsem, gsem, ssem):
    c = lax.axis_index("c")
    s = lax.axis_index("s")
    wid = s * NC + c
    pltpu.sync_copy(z, acc.at[pl.ds(s * RT, RT)])
    plsc.subcore_barrier()
    _prop_body(hs3, se, de, acc, out.at[c], sst, dst_, sidx, didx, buf,
               stsem, gsem, ssem, wid * CPT_HALF, CPT_HALF, s)


@functools.partial(
    pl.kernel,
    out_type=jax.ShapeDtypeStruct((NC, NPAD, HH), jnp.float32),
    mesh=plsc.VectorSubcoreMesh(**_MESH),
    scratch_types=[
        pltpu.VMEM((2, 1, CHUNK), jnp.int32),
        pltpu.VMEM((NGR, DG), jnp.int32),
        pltpu.VMEM((DG, HH), jnp.float32),
        pltpu.VMEM_SHARED((NPAD, HH), jnp.float32),
        pltpu.SemaphoreType.DMA,
        pltpu.SemaphoreType.DMA,
        pltpu.SemaphoreType.DMA,
    ],
)
def _deg_sc(de, ones_h, z16, out, dst_, didx, buf, acc, st0, st1, ssem):
    c = lax.axis_index("c")
    s = lax.axis_index("s")
    wid = s * NC + c
    base = wid * CPT_HALF
    stsems = (st0, st1)
    pltpu.sync_copy(ones_h, buf)
    pltpu.sync_copy(z16, acc.at[pl.ds(s * RT, RT)])
    plsc.subcore_barrier()
    for b in range(2):
        pltpu.async_copy(de.at[base + b], dst_.at[b], stsems[b])

    def chunk2(i, carry):
        for b in range(2):
            ch = i * 2 + b
            pltpu.make_async_copy(de.at[base + b], dst_.at[b], stsems[b]).wait()

            @pl.when(ch > 0)
            def _():
                for g in range(NGR):
                    pltpu.make_async_copy(buf, acc.at[didx.at[g]], ssem).wait()

            for g in range(NGR):
                for j in range(DG // 16):
                    didx[g, pl.ds(j * 16, 16)] = dst_[b, 0, pl.ds(g * DG + j * 16, 16)]

            @pl.when(ch + 2 < CPT_HALF)
            def _():
                pltpu.async_copy(de.at[base + ch + 2], dst_.at[b], stsems[b])

            for g in range(NGR):
                pltpu.async_copy(buf, acc.at[didx.at[g]], ssem, add=True)
        return carry

    lax.fori_loop(0, CPT_HALF // 2, chunk2, 0)
    for g in range(NGR):
        pltpu.make_async_copy(buf, acc.at[didx.at[g]], ssem).wait()
    plsc.subcore_barrier()
    pltpu.sync_copy(acc.at[pl.ds(s * RT, RT)], out.at[c].at[pl.ds(s * RT, RT)])


BM = 1000


def _scalex_body(x_ref, d0_ref, d1_ref, xs_ref, dinv_ref):
    dinv = lax.rsqrt(d0_ref[...] + d1_ref[...] + 1.0)
    xs_ref[...] = x_ref[...] * dinv
    dinv_ref[...] = dinv


_scale_x = pl.pallas_call(
    _scalex_body,
    grid=(N // BM,),
    in_specs=[
        pl.BlockSpec((BM, IN), lambda i: (i, 0)),
        pl.BlockSpec((BM, 1), lambda i: (i, 0)),
        pl.BlockSpec((BM, 1), lambda i: (i, 0)),
    ],
    out_specs=[
        pl.BlockSpec((BM, IN), lambda i: (i, 0)),
        pl.BlockSpec((BM, 1), lambda i: (i, 0)),
    ],
    out_shape=[
        jax.ShapeDtypeStruct((N, IN), jnp.float32),
        jax.ShapeDtypeStruct((N, 1), jnp.float32),
    ],
)


_ACCSPEC = pl.BlockSpec((NC, BM, HH), lambda i: (0, i, 0))


def _l1_body(a_ref, xs_ref, dinv_ref, b_ref, g_ref, be_ref, w1_ref, w2_ref,
             hsn_ref):
    dinv = dinv_ref[...]
    aggx = dinv * (a_ref[0] + a_ref[1] + xs_ref[...])
    pre = jnp.dot(aggx, w1_ref[...], preferred_element_type=jnp.float32) + b_ref[...]
    hact = jnp.maximum(pre * g_ref[...] + be_ref[...], 0.0)
    hsn = jnp.dot(hact, w2_ref[...], preferred_element_type=jnp.float32) * dinv
    hsn_ref[0, :, :] = hsn[:, :HH]
    hsn_ref[1, :, :] = hsn[:, HH:]


_l1 = pl.pallas_call(
    _l1_body,
    grid=(N // BM,),
    in_specs=[
        _ACCSPEC,
        pl.BlockSpec((BM, IN), lambda i: (i, 0)),
        pl.BlockSpec((BM, 1), lambda i: (i, 0)),
        pl.BlockSpec((1, H), lambda i: (0, 0)),
        pl.BlockSpec((1, H), lambda i: (0, 0)),
        pl.BlockSpec((1, H), lambda i: (0, 0)),
        pl.BlockSpec((IN, H), lambda i: (0, 0)),
        pl.BlockSpec((H, H), lambda i: (0, 0)),
    ],
    out_specs=pl.BlockSpec((NC, BM, HH), lambda i: (0, i, 0)),
    out_shape=jax.ShapeDtypeStruct((NC, N, HH), jnp.float32),
)


def _mid3_body(a_ref, hs_ref, dinv_ref, b_ref, g_ref, be_ref, w_ref,
               h_ref, hsn_ref):
    dinv = dinv_ref[...]
    pre = jnp.concatenate(
        [a_ref[0] + hs_ref[0], a_ref[1] + hs_ref[1]], axis=1)
    pre = dinv * pre + b_ref[...]
    hact = jnp.maximum(pre * g_ref[...] + be_ref[...], 0.0)
    h_ref[...] = hact
    hsn_ref[...] = jnp.dot(hact, w_ref[...], preferred_element_type=jnp.float32) * dinv


_mid3 = pl.pallas_call(
    _mid3_body,
    grid=(N // BM,),
    in_specs=[
        _ACCSPEC,
        _ACCSPEC,
        pl.BlockSpec((BM, 1), lambda i: (i, 0)),
        pl.BlockSpec((1, H), lambda i: (0, 0)),
        pl.BlockSpec((1, H), lambda i: (0, 0)),
        pl.BlockSpec((1, H), lambda i: (0, 0)),
        pl.BlockSpec((H, HH), lambda i: (0, 0)),
    ],
    out_specs=[
        pl.BlockSpec((BM, H), lambda i: (i, 0)),
        pl.BlockSpec((BM, HH), lambda i: (i, 0)),
    ],
    out_shape=[
        jax.ShapeDtypeStruct((N, H), jnp.float32),
        jax.ShapeDtypeStruct((N, HH), jnp.float32),
    ],
)


def _out_body(o_ref, hs_ref, dinv_ref, b_ref, out_ref):
    t = dinv_ref[...] * (o_ref[0] + o_ref[1] + hs_ref[...]) + b_ref[...]
    col = lax.broadcasted_iota(jnp.int32, t.shape, 1)
    valid = col < OUT
    tm = jnp.where(valid, t, -jnp.inf)
    mx = jnp.max(tm, axis=1, keepdims=True)
    ex = jnp.where(valid, jnp.exp(t - mx), 0.0)
    lse = jnp.log(jnp.sum(ex, axis=1, keepdims=True)) + mx
    out_ref[...] = t - lse


_tc_out = pl.pallas_call(
    _out_body,
    grid=(N // BM,),
    in_specs=[
        _ACCSPEC,
        pl.BlockSpec((BM, HH), lambda i: (i, 0)),
        pl.BlockSpec((BM, 1), lambda i: (i, 0)),
        pl.BlockSpec((1, HH), lambda i: (0, 0)),
    ],
    out_specs=pl.BlockSpec((BM, HH), lambda i: (i, 0)),
    out_shape=jax.ShapeDtypeStruct((N, HH), jnp.float32),
)


def kernel(x, edge_index, W1, b1, g1, be1, W2, b2, g2, be2, W3, b3):
    f32 = jnp.float32
    pidx = jnp.arange(E2 - E, dtype=jnp.int32)
    se = jnp.concatenate([edge_index[0], pidx % N])
    de = jnp.concatenate([edge_index[1], N + pidx % (NPAD - N)])
    se = se.reshape(NCHT, 1, CHUNK)
    de = de.reshape(NCHT, 1, CHUNK)

    z = jnp.zeros((RT, HH), f32)
    ones_h = jnp.ones((DG, HH), f32)

    dego = _deg_sc(de, ones_h, z)
    d0 = dego[0, :N, 0:1]
    d1 = dego[1, :N, 0:1]
    xs, dinv = _scale_x(x, d0, d1)

    accx = _prop_half(xs, se, de, z)
    hs2 = _l1(accx, xs, dinv,
              b1.reshape(1, H), g1.reshape(1, H), be1.reshape(1, H), W1, W2)
    acc2 = _prop_col(hs2, se, de, z)
    W3p = jnp.pad(W3, ((0, 0), (0, HH - OUT)))
    h2, hs3 = _mid3(acc2, hs2, dinv,
                    b2.reshape(1, H), g2.reshape(1, H), be2.reshape(1, H), W3p)
    acc3 = _prop_half(hs3, se, de, z)
    b3p = jnp.pad(b3, (0, HH - OUT)).reshape(1, HH)
    outp = _tc_out(acc3, hs3, dinv, b3p)
    return outp[:, :OUT], h2


# speedup vs baseline: 1.0058x; 1.0058x over previous
import functools

import jax
import jax.numpy as jnp
from jax import lax
from jax.experimental import pallas as pl
from jax.experimental.pallas import tpu as pltpu
from jax.experimental.pallas import tpu_sc as plsc

N = 10000
E = 320000
IN = 128
H = 256
HH = 128
OUT = 40

NC = 2
NS = 16
NPAD = 10240
DUMP = NPAD - 1
CHUNK = 320
G = 80
NSL = CHUNK // G
DG = 80
NGR = CHUNK // DG
E2 = 327680
NCHT = E2 // CHUNK
CPT_COL = NCHT // NS
CPT_HALF = NCHT // (NC * NS)
RT = NPAD // NS

_MESH = dict(core_axis_name="c", subcore_axis_name="s")


def _prop_body(tab, se, de, acc, out_c, sst, dst_, sidx, didx, buf,
               stsem, gsem, ssem, base, cpt, s):
    for b in range(2):
        pltpu.async_copy(se.at[base + b], sst.at[b], stsem.at[b])
        pltpu.async_copy(de.at[base + b], dst_.at[b], stsem.at[b])

    def chunk2(i, carry):
        for b in range(2):
            ch = i * 2 + b
            pltpu.make_async_copy(se.at[base + b], sst.at[b], stsem.at[b]).wait()
            pltpu.make_async_copy(de.at[base + b], dst_.at[b], stsem.at[b]).wait()

            for r in range(NSL):
                ds_ = b * NSL + r
                for j in range(G // 16):
                    sidx[r, pl.ds(j * 16, 16)] = sst[b, 0, pl.ds(r * G + j * 16, 16)]
                    didx[ds_, pl.ds(j * 16, 16)] = dst_[b, 0, pl.ds(r * G + j * 16, 16)]

                @pl.when(ch > 0)
                def _():
                    pltpu.make_async_copy(
                        buf.at[r], acc.at[didx.at[ds_]], ssem.at[r]).wait()

                pltpu.async_copy(tab.at[sidx.at[r]], buf.at[r], gsem.at[r])

            @pl.when(ch + 2 < cpt)
            def _():
                pltpu.async_copy(se.at[base + ch + 2], sst.at[b], stsem.at[b])
                pltpu.async_copy(de.at[base + ch + 2], dst_.at[b], stsem.at[b])

            for r in range(NSL):
                ds_ = b * NSL + r
                pltpu.make_async_copy(tab.at[sidx.at[r]], buf.at[r], gsem.at[r]).wait()
                pltpu.async_copy(buf.at[r], acc.at[didx.at[ds_]], ssem.at[r], add=True)
        return carry

    lax.fori_loop(0, cpt // 2, chunk2, 0)
    for r in range(NSL):
        pltpu.make_async_copy(buf.at[r], acc.at[didx.at[NSL + r]], ssem.at[r]).wait()
    plsc.subcore_barrier()
    pltpu.sync_copy(acc.at[pl.ds(s * RT, RT)], out_c.at[pl.ds(s * RT, RT)])


def _prop_scratch():
    return [
        pltpu.VMEM((2, 1, CHUNK), jnp.int32),
        pltpu.VMEM((2, 1, CHUNK), jnp.int32),
        pltpu.VMEM((NSL, G), jnp.int32),
        pltpu.VMEM((2 * NSL, G), jnp.int32),
        pltpu.VMEM((NSL, G, HH), jnp.float32),
        pltpu.VMEM_SHARED((NPAD, HH), jnp.float32),
        pltpu.SemaphoreType.DMA((2,)),
        pltpu.SemaphoreType.DMA((NSL,)),
        pltpu.SemaphoreType.DMA((NSL,)),
    ]


@functools.partial(
    pl.kernel,
    out_type=jax.ShapeDtypeStruct((NC, NPAD, HH), jnp.float32),
    mesh=plsc.VectorSubcoreMesh(**_MESH),
    scratch_types=_prop_scratch(),
)
def _prop_col(hs2, se, de, z, out, sst, dst_, sidx, didx, buf, acc,
              stsem, gsem, ssem):
    c = lax.axis_index("c")
    s = lax.axis_index("s")
    pltpu.sync_copy(z, acc.at[pl.ds(s * RT, RT)])
    plsc.subcore_barrier()
    _prop_body(hs2.at[c], se, de, acc, out.at[c], sst, dst_, sidx, didx, buf,
               stsem, gsem, ssem, s * CPT_COL, CPT_COL, s)


@functools.partial(
    pl.kernel,
    out_type=jax.ShapeDtypeStruct((NC, NPAD, HH), jnp.float32),
    mesh=plsc.VectorSubcoreMesh(**_MESH),
    scratch_types=_prop_scratch(),
)
def _prop_half(hs3, se, de, z, out, sst, dst_, sidx, didx, buf, acc,
               stsem, gsem, ssem):
    c = lax.axis_index("c")
    s = lax.axis_index("s")
    wid = s * NC + c
    pltpu.sync_copy(z, acc.at[pl.ds(s * RT, RT)])
    plsc.subcore_barrier()
    _prop_body(hs3, se, de, acc, out.at[c], sst, dst_, sidx, didx, buf,
               stsem, gsem, ssem, wid * CPT_HALF, CPT_HALF, s)


@functools.partial(
    pl.kernel,
    out_type=jax.ShapeDtypeStruct((NC, NPAD, HH), jnp.float32),
    mesh=plsc.VectorSubcoreMesh(**_MESH),
    scratch_types=[
        pltpu.VMEM((2, 1, CHUNK), jnp.int32),
        pltpu.VMEM((NGR, DG), jnp.int32),
        pltpu.VMEM((DG, HH), jnp.float32),
        pltpu.VMEM_SHARED((NPAD, HH), jnp.float32),
        pltpu.SemaphoreType.DMA,
        pltpu.SemaphoreType.DMA,
        pltpu.SemaphoreType.DMA,
    ],
)
def _deg_sc(de, ones_h, z16, out, dst_, didx, buf, acc, st0, st1, ssem):
    c = lax.axis_index("c")
    s = lax.axis_index("s")
    wid = s * NC + c
    base = wid * CPT_HALF
    stsems = (st0, st1)
    pltpu.sync_copy(ones_h, buf)
    pltpu.sync_copy(z16, acc.at[pl.ds(s * RT, RT)])
    plsc.subcore_barrier()
    for b in range(2):
        pltpu.async_copy(de.at[base + b], dst_.at[b], stsems[b])

    def chunk2(i, carry):
        for b in range(2):
            ch = i * 2 + b
            pltpu.make_async_copy(de.at[base + b], dst_.at[b], stsems[b]).wait()

            @pl.when(ch > 0)
            def _():
                for g in range(NGR):
                    pltpu.make_async_copy(buf, acc.at[didx.at[g]], ssem).wait()

            for g in range(NGR):
                for j in range(DG // 16):
                    didx[g, pl.ds(j * 16, 16)] = dst_[b, 0, pl.ds(g * DG + j * 16, 16)]

            @pl.when(ch + 2 < CPT_HALF)
            def _():
                pltpu.async_copy(de.at[base + ch + 2], dst_.at[b], stsems[b])

            for g in range(NGR):
                pltpu.async_copy(buf, acc.at[didx.at[g]], ssem, add=True)
        return carry

    lax.fori_loop(0, CPT_HALF // 2, chunk2, 0)
    for g in range(NGR):
        pltpu.make_async_copy(buf, acc.at[didx.at[g]], ssem).wait()
    plsc.subcore_barrier()
    pltpu.sync_copy(acc.at[pl.ds(s * RT, RT)], out.at[c].at[pl.ds(s * RT, RT)])


BM = 1000


def _scalex_body(x_ref, d0_ref, d1_ref, xs_ref, dinv_ref):
    dinv = lax.rsqrt(d0_ref[...] + d1_ref[...] + 1.0)
    xs_ref[...] = x_ref[...] * dinv
    dinv_ref[...] = dinv


_scale_x = pl.pallas_call(
    _scalex_body,
    grid=(N // BM,),
    in_specs=[
        pl.BlockSpec((BM, IN), lambda i: (i, 0)),
        pl.BlockSpec((BM, 1), lambda i: (i, 0)),
        pl.BlockSpec((BM, 1), lambda i: (i, 0)),
    ],
    out_specs=[
        pl.BlockSpec((BM, IN), lambda i: (i, 0)),
        pl.BlockSpec((BM, 1), lambda i: (i, 0)),
    ],
    out_shape=[
        jax.ShapeDtypeStruct((N, IN), jnp.float32),
        jax.ShapeDtypeStruct((N, 1), jnp.float32),
    ],
)


_ACCSPEC = pl.BlockSpec((NC, BM, HH), lambda i: (0, i, 0))


def _l1_body(a_ref, xs_ref, dinv_ref, b_ref, g_ref, be_ref, w1_ref, w2_ref,
             hsn_ref):
    dinv = dinv_ref[...]
    aggx = dinv * (a_ref[0] + a_ref[1] + xs_ref[...])
    pre = jnp.dot(aggx, w1_ref[...], preferred_element_type=jnp.float32) + b_ref[...]
    hact = jnp.maximum(pre * g_ref[...] + be_ref[...], 0.0)
    hsn = jnp.dot(hact, w2_ref[...], preferred_element_type=jnp.float32) * dinv
    hsn_ref[0, :, :] = hsn[:, :HH]
    hsn_ref[1, :, :] = hsn[:, HH:]


_l1 = pl.pallas_call(
    _l1_body,
    grid=(N // BM,),
    in_specs=[
        _ACCSPEC,
        pl.BlockSpec((BM, IN), lambda i: (i, 0)),
        pl.BlockSpec((BM, 1), lambda i: (i, 0)),
        pl.BlockSpec((1, H), lambda i: (0, 0)),
        pl.BlockSpec((1, H), lambda i: (0, 0)),
        pl.BlockSpec((1, H), lambda i: (0, 0)),
        pl.BlockSpec((IN, H), lambda i: (0, 0)),
        pl.BlockSpec((H, H), lambda i: (0, 0)),
    ],
    out_specs=pl.BlockSpec((NC, BM, HH), lambda i: (0, i, 0)),
    out_shape=jax.ShapeDtypeStruct((NC, N, HH), jnp.float32),
)


def _mid3_body(a_ref, hs_ref, dinv_ref, b_ref, g_ref, be_ref, w_ref,
               h_ref, hsn_ref):
    dinv = dinv_ref[...]
    pre = jnp.concatenate(
        [a_ref[0] + hs_ref[0], a_ref[1] + hs_ref[1]], axis=1)
    pre = dinv * pre + b_ref[...]
    hact = jnp.maximum(pre * g_ref[...] + be_ref[...], 0.0)
    h_ref[...] = hact
    hsn_ref[...] = jnp.dot(hact, w_ref[...], preferred_element_type=jnp.float32) * dinv


_mid3 = pl.pallas_call(
    _mid3_body,
    grid=(N // BM,),
    in_specs=[
        _ACCSPEC,
        _ACCSPEC,
        pl.BlockSpec((BM, 1), lambda i: (i, 0)),
        pl.BlockSpec((1, H), lambda i: (0, 0)),
        pl.BlockSpec((1, H), lambda i: (0, 0)),
        pl.BlockSpec((1, H), lambda i: (0, 0)),
        pl.BlockSpec((H, HH), lambda i: (0, 0)),
    ],
    out_specs=[
        pl.BlockSpec((BM, H), lambda i: (i, 0)),
        pl.BlockSpec((BM, HH), lambda i: (i, 0)),
    ],
    out_shape=[
        jax.ShapeDtypeStruct((N, H), jnp.float32),
        jax.ShapeDtypeStruct((N, HH), jnp.float32),
    ],
)


def _out_body(o_ref, hs_ref, dinv_ref, b_ref, out_ref):
    t = dinv_ref[...] * (o_ref[0] + o_ref[1] + hs_ref[...]) + b_ref[...]
    col = lax.broadcasted_iota(jnp.int32, t.shape, 1)
    valid = col < OUT
    tm = jnp.where(valid, t, -jnp.inf)
    mx = jnp.max(tm, axis=1, keepdims=True)
    ex = jnp.where(valid, jnp.exp(t - mx), 0.0)
    lse = jnp.log(jnp.sum(ex, axis=1, keepdims=True)) + mx
    out_ref[...] = t - lse


_tc_out = pl.pallas_call(
    _out_body,
    grid=(N // BM,),
    in_specs=[
        _ACCSPEC,
        pl.BlockSpec((BM, HH), lambda i: (i, 0)),
        pl.BlockSpec((BM, 1), lambda i: (i, 0)),
        pl.BlockSpec((1, HH), lambda i: (0, 0)),
    ],
    out_specs=pl.BlockSpec((BM, HH), lambda i: (i, 0)),
    out_shape=jax.ShapeDtypeStruct((N, HH), jnp.float32),
)


def kernel(x, edge_index, W1, b1, g1, be1, W2, b2, g2, be2, W3, b3):
    f32 = jnp.float32
    pidx = jnp.arange(E2 - E, dtype=jnp.int32)
    se = jnp.concatenate([edge_index[0], pidx % N])
    de = jnp.concatenate([edge_index[1], N + pidx % (NPAD - N)])
    se = se.reshape(NCHT, 1, CHUNK)
    de = de.reshape(NCHT, 1, CHUNK)

    z = jnp.zeros((RT, HH), f32)
    ones_h = jnp.ones((DG, HH), f32)

    dego = _deg_sc(de, ones_h, z)
    d0 = dego[0, :N, 0:1]
    d1 = dego[1, :N, 0:1]
    xs, dinv = _scale_x(x, d0, d1)

    accx = _prop_half(xs, se, de, z)
    hs2 = _l1(accx, xs, dinv,
              b1.reshape(1, H), g1.reshape(1, H), be1.reshape(1, H), W1, W2)
    acc2 = _prop_col(hs2, se, de, z)
    W3p = jnp.pad(W3, ((0, 0), (0, HH - OUT)))
    h2, hs3 = _mid3(acc2, hs2, dinv,
                    b2.reshape(1, H), g2.reshape(1, H), be2.reshape(1, H), W3p)
    acc3 = _prop_half(hs3, se, de, z)
    b3p = jnp.pad(b3, (0, HH - OUT)).reshape(1, HH)
    outp = _tc_out(acc3, hs3, dinv, b3p)
    return outp[:, :OUT], h2
